# Initial kernel scaffold; baseline (speedup 1.0000x reference)
import jax
import jax.numpy as jnp
from jax.experimental import pallas as pl


def kernel(h0, h1, h2, label, im_dimx, im_dimy):
    raise NotImplementedError("write your pallas kernel here")


# speedup vs baseline: 2.8861x; 2.8861x over previous
import jax
import jax.numpy as jnp
from jax.experimental import pallas as pl

_NUM_BOXES_PADDED = 128


def _lamm_body(h0_ref, h1_ref, h2_ref, lab_ref, dims_ref, out_ref):
    dimx = dims_ref[0, 0]
    dimy = dims_ref[0, 1]
    lab = lab_ref[:, :]
    bx1 = lab[:, 0:1]
    by1 = lab[:, 1:2]
    bx2 = lab[:, 2:3]
    by2 = lab[:, 3:4]

    total = jnp.float32(0.0)
    for h_ref, (n, hgt, wid) in (
        (h0_ref, (8, 200, 336)),
        (h1_ref, (8, 100, 168)),
        (h2_ref, (8, 50, 84)),
    ):
        sx = wid / dimx
        sy = hgt / dimy
        x1 = jnp.clip(jnp.round(bx1 * sx), 0.0, wid - 1.0)
        y1 = jnp.clip(jnp.round(by1 * sy), 0.0, hgt - 1.0)
        x2 = jnp.clip(jnp.round(bx2 * sx), 0.0, float(wid))
        y2 = jnp.clip(jnp.round(by2 * sy), 0.0, float(hgt))
        valid = ((x2 > x1) & (y2 > y1)).astype(jnp.float32)

        xx = jax.lax.broadcasted_iota(
            jnp.int32, (_NUM_BOXES_PADDED, wid), 1).astype(jnp.float32)
        yy = jax.lax.broadcasted_iota(
            jnp.int32, (_NUM_BOXES_PADDED, hgt), 1).astype(jnp.float32)
        xm = ((xx >= x1) & (xx < x2)).astype(jnp.float32) * valid
        ym = ((yy >= y1) & (yy < y2)).astype(jnp.float32)
        cov = jax.lax.dot_general(
            ym, xm, (((0,), (0,)), ((), ())),
            preferred_element_type=jnp.float32,
        )
        area = jnp.sum((cov > 0.5).astype(jnp.float32))

        s = jnp.sum(h_ref[:, :])
        tn = float(n * hgt * wid)
        li = (s / tn - area / float(hgt * wid)) ** 2
        total = total + li

    out_ref[:, :] = jnp.reshape(total / 3.0, (1, 1))


def kernel(h0, h1, h2, label, im_dimx, im_dimy):
    h0f = h0.reshape(8 * 200, 336)
    h1f = h1.reshape(8 * 100, 168)
    h2f = h2.reshape(8 * 50, 84)
    lab = jnp.pad(label.astype(jnp.float32),
                  ((0, _NUM_BOXES_PADDED - label.shape[0]), (0, 0)))
    dims = jnp.stack([jnp.asarray(im_dimx, jnp.float32),
                      jnp.asarray(im_dimy, jnp.float32)]).reshape(1, 2)
    out = pl.pallas_call(
        _lamm_body,
        out_shape=jax.ShapeDtypeStruct((1, 1), jnp.float32),
    )(h0f, h1f, h2f, lab, dims)
    return out.reshape(())
